# baseline (speedup 1.0000x reference)
import jax
import jax.numpy as jnp
from jax.experimental import pallas as pl
from jax.experimental.pallas import tpu as pltpu
from functools import partial

_THRESHOLD = 5.0
_KH, _KW = 3, 3


def _cbconv_body(x_ref, pi_ref, po_ref, wf_ref, br_ref, out_ref,
                 xpad_ref, cpad_ref, rhs_ref, l_ref, bcol_ref, mlr_ref,
                 *, C, Cout, H, W, Rb, PAD, NB):
    HW = H * W
    Nb = Rb * W
    NW = Nb + 256
    b = pl.program_id(0)
    j = pl.program_id(1)

    @pl.when(jnp.logical_and(b == 0, j == 0))
    def _init_call():
        f_ids = jax.lax.broadcasted_iota(jnp.int32, (9 * C, 3 * C), 0)
        s_ids = jax.lax.broadcasted_iota(jnp.int32, (9 * C, 3 * C), 1)
        f_target = (s_ids % C) * 9 + (s_ids // C) * 3
        for dw in range(3):
            q = (f_ids == f_target + dw).astype(jnp.float32)
            l_ref[dw * Cout:(dw + 1) * Cout, :] = jnp.dot(
                wf_ref[:, :], q, preferred_element_type=jnp.float32)
        bcol_ref[:, :] = jnp.transpose(br_ref[:, :], (1, 0))
        lane = jax.lax.broadcasted_iota(jnp.int32, (1, Nb), 1) % W
        mlr_ref[0:1, :] = (lane != 0).astype(jnp.float32)
        mlr_ref[1:2, :] = (lane != (W - 1)).astype(jnp.float32)
        xpad_ref[:, 0:PAD] = jnp.zeros((C, PAD), jnp.float32)
        xpad_ref[:, PAD + HW:] = jnp.zeros((C, PAD), jnp.float32)
        z1 = jnp.zeros((1, PAD), jnp.float32)
        cpad_ref[:, 0:PAD] = z1
        cpad_ref[:, PAD + HW:] = z1

    @pl.when(j < NB)
    def _stage_chunk():
        xc = x_ref[0]
        xpad_ref[:, pl.ds(PAD + j * Nb, Nb)] = xc.reshape(C, Nb)
        m = jnp.max(jnp.abs(xc - pi_ref[0]), axis=0)
        cpad_ref[:, pl.ds(PAD + j * Nb, Nb)] = \
            (m > _THRESHOLD).astype(jnp.float32).reshape(1, Nb)

    @pl.when(j > 0)
    def _compute_block():
        rb = j - 1
        mleft = mlr_ref[0:1, :]
        mright = mlr_ref[1:2, :]

        base = PAD + rb * Nb
        for dh in range(3):
            rhs_ref[dh * C:(dh + 1) * C, :] = \
                xpad_ref[:, pl.ds(base + (dh - 1) * W - 128, NW)]

        z = jnp.dot(l_ref[:, :], rhs_ref[:, :],
                    preferred_element_type=jnp.float32)
        y = (jax.lax.slice(z, (0, 127), (Cout, 127 + Nb)) * mleft
             + jax.lax.slice(z, (Cout, 128), (2 * Cout, 128 + Nb))
             + jax.lax.slice(z, (2 * Cout, 129), (3 * Cout, 129 + Nb))
             * mright
             + bcol_ref[:, :])

        dil = None
        for dr in range(3):
            cw = cpad_ref[:, pl.ds(base + (dr - 1) * W - 128, NW)]
            for dc in range(3):
                s = jax.lax.slice(cw, (0, 128 + dc - 1),
                                  (1, 128 + dc - 1 + Nb))
                if dc == 0:
                    s = s * mleft
                elif dc == 2:
                    s = s * mright
                dil = s if dil is None else jnp.maximum(dil, s)

        sel = jnp.where(dil > 0.0, y, po_ref[0].reshape(Cout, Nb))
        for t in range(Rb // 8):
            out_ref[0, :, t * 8:(t + 1) * 8, :] = jax.lax.slice(
                sel, (0, t * 8 * W), (Cout, (t + 1) * 8 * W)
            ).reshape(Cout, 8, W)


def kernel(x, prev_input, prev_output, weight, bias):
    B, C, H, W = x.shape
    Cout = weight.shape[0]
    HW = H * W
    Rb = min(64, H)
    NB = H // Rb
    Nb = Rb * W
    PAD = 256

    wf = weight.reshape(Cout, C * _KH * _KW)
    br = bias.reshape(1, Cout)

    body = partial(_cbconv_body, C=C, Cout=Cout, H=H, W=W, Rb=Rb, PAD=PAD,
                   NB=NB)
    last = NB - 1
    out = pl.pallas_call(
        body,
        grid=(B, NB + 1),
        in_specs=[
            pl.BlockSpec((1, C, Rb, W),
                         lambda b, j: (b, 0, jnp.minimum(j, last), 0)),
            pl.BlockSpec((1, C, Rb, W),
                         lambda b, j: (b, 0, jnp.minimum(j, last), 0)),
            pl.BlockSpec((1, Cout, Rb, W),
                         lambda b, j: (b, 0, jnp.maximum(j - 1, 0), 0)),
            pl.BlockSpec((Cout, _KH * _KW * C), lambda b, j: (0, 0)),
            pl.BlockSpec((1, Cout), lambda b, j: (0, 0)),
        ],
        out_specs=pl.BlockSpec((1, Cout, Rb, W),
                               lambda b, j: (b, 0, jnp.maximum(j - 1, 0), 0)),
        out_shape=jax.ShapeDtypeStruct((B, Cout, H, W), jnp.float32),
        scratch_shapes=[
            pltpu.VMEM((C, PAD + HW + PAD), jnp.float32),
            pltpu.VMEM((1, PAD + HW + PAD), jnp.float32),
            pltpu.VMEM((3 * C, Nb + 256), jnp.float32),
            pltpu.VMEM((3 * Cout, 3 * C), jnp.float32),
            pltpu.VMEM((Cout, 1), jnp.float32),
            pltpu.VMEM((2, Nb), jnp.float32),
        ],
    )(x, prev_input, prev_output, wf, br)
    return out


# speedup vs baseline: 1.0059x; 1.0059x over previous
import jax
import jax.numpy as jnp
from jax.experimental import pallas as pl
from jax.experimental.pallas import tpu as pltpu
from functools import partial

_THRESHOLD = 5.0
_KH, _KW = 3, 3


def _cbconv_body(x_ref, pi_ref, po_ref, wf_ref, br_ref, out_ref,
                 xpad_ref, cpad_ref, rhs_ref, l_ref, bcol_ref,
                 *, C, Cout, H, W, Rb, PAD, NB):
    HW = H * W
    Nb = Rb * W
    NW = Nb + 256
    b = pl.program_id(0)
    j = pl.program_id(1)

    @pl.when(jnp.logical_and(b == 0, j == 0))
    def _init_call():
        f_ids = jax.lax.broadcasted_iota(jnp.int32, (9 * C, 3 * C), 0)
        s_ids = jax.lax.broadcasted_iota(jnp.int32, (9 * C, 3 * C), 1)
        f_target = (s_ids % C) * 9 + (s_ids // C) * 3
        for dw in range(3):
            q = (f_ids == f_target + dw).astype(jnp.float32)
            l_ref[dw * Cout:(dw + 1) * Cout, :] = jnp.dot(
                wf_ref[:, :], q, preferred_element_type=jnp.float32)
        bcol_ref[:, :] = jnp.transpose(br_ref[:, :], (1, 0))
        xpad_ref[:, 0:PAD] = jnp.zeros((C, PAD), jnp.float32)
        xpad_ref[:, PAD + HW:] = jnp.zeros((C, PAD), jnp.float32)
        z1 = jnp.zeros((1, PAD), jnp.float32)
        cpad_ref[:, 0:PAD] = z1
        cpad_ref[:, PAD + HW:] = z1

    @pl.when(j < NB)
    def _stage_chunk():
        xc = x_ref[0]
        xpad_ref[:, pl.ds(PAD + j * Nb, Nb)] = xc.reshape(C, Nb)
        m = jnp.max(jnp.abs(xc - pi_ref[0]), axis=0)
        cpad_ref[:, pl.ds(PAD + j * Nb, Nb)] = \
            (m > _THRESHOLD).astype(jnp.float32).reshape(1, Nb)

    @pl.when(j > 0)
    def _compute_block():
        rb = j - 1
        lane = jax.lax.broadcasted_iota(jnp.int32, (1, Nb), 1) % W
        mleft = (lane != 0).astype(jnp.float32)
        mright = (lane != (W - 1)).astype(jnp.float32)

        base = PAD + rb * Nb
        for dh in range(3):
            rhs_ref[dh * C:(dh + 1) * C, :] = \
                xpad_ref[:, pl.ds(base + (dh - 1) * W - 128, NW)]

        z = jnp.dot(l_ref[:, :], rhs_ref[:, :],
                    preferred_element_type=jnp.float32)
        y = (jax.lax.slice(z, (0, 127), (Cout, 127 + Nb)) * mleft
             + jax.lax.slice(z, (Cout, 128), (2 * Cout, 128 + Nb))
             + jax.lax.slice(z, (2 * Cout, 129), (3 * Cout, 129 + Nb))
             * mright
             + bcol_ref[:, :])

        dil = None
        for dr in range(3):
            cw = cpad_ref[:, pl.ds(base + (dr - 1) * W - 128, NW)]
            for dc in range(3):
                s = jax.lax.slice(cw, (0, 128 + dc - 1),
                                  (1, 128 + dc - 1 + Nb))
                if dc == 0:
                    s = s * mleft
                elif dc == 2:
                    s = s * mright
                dil = s if dil is None else jnp.maximum(dil, s)

        sel = jnp.where(dil > 0.0, y, po_ref[0].reshape(Cout, Nb))
        for t in range(Rb // 8):
            out_ref[0, :, t * 8:(t + 1) * 8, :] = jax.lax.slice(
                sel, (0, t * 8 * W), (Cout, (t + 1) * 8 * W)
            ).reshape(Cout, 8, W)


def kernel(x, prev_input, prev_output, weight, bias):
    B, C, H, W = x.shape
    Cout = weight.shape[0]
    HW = H * W
    Rb = min(64, H)
    NB = H // Rb
    Nb = Rb * W
    PAD = 256

    wf = weight.reshape(Cout, C * _KH * _KW)
    br = bias.reshape(1, Cout)

    body = partial(_cbconv_body, C=C, Cout=Cout, H=H, W=W, Rb=Rb, PAD=PAD,
                   NB=NB)
    last = NB - 1
    out = pl.pallas_call(
        body,
        grid=(B, NB + 1),
        in_specs=[
            pl.BlockSpec((1, C, Rb, W),
                         lambda b, j: (b, 0, jnp.minimum(j, last), 0)),
            pl.BlockSpec((1, C, Rb, W),
                         lambda b, j: (b, 0, jnp.minimum(j, last), 0)),
            pl.BlockSpec((1, Cout, Rb, W),
                         lambda b, j: (b, 0, jnp.maximum(j - 1, 0), 0)),
            pl.BlockSpec((Cout, _KH * _KW * C), lambda b, j: (0, 0)),
            pl.BlockSpec((1, Cout), lambda b, j: (0, 0)),
        ],
        out_specs=pl.BlockSpec((1, Cout, Rb, W),
                               lambda b, j: (b, 0, jnp.maximum(j - 1, 0), 0)),
        out_shape=jax.ShapeDtypeStruct((B, Cout, H, W), jnp.float32),
        scratch_shapes=[
            pltpu.VMEM((C, PAD + HW + PAD), jnp.float32),
            pltpu.VMEM((1, PAD + HW + PAD), jnp.float32),
            pltpu.VMEM((3 * C, Nb + 256), jnp.float32),
            pltpu.VMEM((3 * Cout, 3 * C), jnp.float32),
            pltpu.VMEM((Cout, 1), jnp.float32),
        ],
    )(x, prev_input, prev_output, wf, br)
    return out
